# baseline (device time: 30335 ns/iter reference)
import jax
import jax.numpy as jnp
from jax import lax
from jax.experimental import pallas as pl
from jax.experimental.pallas import tpu as pltpu

N_DEV = 4
N_GLOBAL = 2048
EPS = 1e-5


def kernel(x, gamma, beta):
    m, n_per = x.shape

    def body(x_ref, gamma_ref, beta_ref, out_ref, comm_ref, send_sems, recv_sems):
        my = lax.axis_index("i")
        left = lax.rem(my + N_DEV - 1, N_DEV)
        right = lax.rem(my + 1, N_DEV)

        barrier_sem = pltpu.get_barrier_semaphore()
        for nbr in [left, right]:
            pl.semaphore_signal(
                barrier_sem, inc=1,
                device_id=(nbr,), device_id_type=pl.DeviceIdType.MESH,
            )
        pl.semaphore_wait(barrier_sem, 2)

        x = x_ref[:, :]
        psum = jnp.sum(x, axis=1, keepdims=True)
        psq = jnp.sum(x * x, axis=1, keepdims=True)
        stats = jnp.concatenate([psum, psq], axis=1)
        comm_ref[0] = stats

        acc = stats
        for h in range(N_DEV - 1):
            rdma = pltpu.make_async_remote_copy(
                src_ref=comm_ref.at[h],
                dst_ref=comm_ref.at[h + 1],
                send_sem=send_sems.at[h],
                recv_sem=recv_sems.at[h],
                device_id=(right,),
                device_id_type=pl.DeviceIdType.MESH,
            )
            rdma.start()
            rdma.wait()
            acc = acc + comm_ref[h + 1]

        mean = acc[:, 0:1] / N_GLOBAL
        var = acc[:, 1:2] / N_GLOBAL - mean * mean
        rstd = lax.rsqrt(var + EPS)
        g = gamma_ref[0:1, :]
        b = beta_ref[0:1, :]
        out_ref[:, :] = g * ((x - mean) * rstd) + b

    return pl.pallas_call(
        body,
        out_shape=jax.ShapeDtypeStruct((m, n_per), x.dtype),
        in_specs=[
            pl.BlockSpec(memory_space=pltpu.VMEM),
            pl.BlockSpec(memory_space=pltpu.VMEM),
            pl.BlockSpec(memory_space=pltpu.VMEM),
        ],
        out_specs=pl.BlockSpec(memory_space=pltpu.VMEM),
        scratch_shapes=[
            pltpu.VMEM((N_DEV, m, 2), jnp.float32),
            pltpu.SemaphoreType.DMA((N_DEV - 1,)),
            pltpu.SemaphoreType.DMA((N_DEV - 1,)),
        ],
        compiler_params=pltpu.CompilerParams(collective_id=0),
    )(x, gamma.reshape(1, n_per), beta.reshape(1, n_per))


# device time: 10593 ns/iter; 2.8637x vs baseline; 2.8637x over previous
import jax
import jax.numpy as jnp
from jax import lax
from jax.experimental import pallas as pl
from jax.experimental.pallas import tpu as pltpu

N_DEV = 4
N_GLOBAL = 2048
EPS = 1e-5


def kernel(x, gamma, beta):
    m, n_per = x.shape

    def body(x_ref, gamma_ref, beta_ref, out_ref, comm_ref, send_sems, recv_sems):
        my = lax.axis_index("i")

        barrier_sem = pltpu.get_barrier_semaphore()
        for d in range(1, N_DEV):
            pl.semaphore_signal(
                barrier_sem, inc=1,
                device_id=(lax.rem(my + d, N_DEV),),
                device_id_type=pl.DeviceIdType.MESH,
            )
        pl.semaphore_wait(barrier_sem, N_DEV - 1)

        x = x_ref[:, :]
        psum = jnp.sum(x, axis=1, keepdims=True)
        psq = jnp.sum(x * x, axis=1, keepdims=True)
        stats = jnp.concatenate([psum, psq], axis=1)
        comm_ref[0] = stats.T

        rdmas = []
        for d in range(1, N_DEV):
            rdma = pltpu.make_async_remote_copy(
                src_ref=comm_ref.at[0],
                dst_ref=comm_ref.at[d],
                send_sem=send_sems.at[d - 1],
                recv_sem=recv_sems.at[d - 1],
                device_id=(lax.rem(my + d, N_DEV),),
                device_id_type=pl.DeviceIdType.MESH,
            )
            rdma.start()
            rdmas.append(rdma)
        for rdma in rdmas:
            rdma.wait_send()
        for rdma in rdmas:
            rdma.wait_recv()

        total = comm_ref[0] + comm_ref[1] + comm_ref[2] + comm_ref[3]
        total = total.T
        mean = total[:, 0:1] / N_GLOBAL
        var = total[:, 1:2] / N_GLOBAL - mean * mean
        rstd = lax.rsqrt(var + EPS)
        g = gamma_ref[0:1, :]
        b = beta_ref[0:1, :]
        out_ref[:, :] = g * ((x - mean) * rstd) + b

    return pl.pallas_call(
        body,
        out_shape=jax.ShapeDtypeStruct((m, n_per), x.dtype),
        in_specs=[
            pl.BlockSpec(memory_space=pltpu.VMEM),
            pl.BlockSpec(memory_space=pltpu.VMEM),
            pl.BlockSpec(memory_space=pltpu.VMEM),
        ],
        out_specs=pl.BlockSpec(memory_space=pltpu.VMEM),
        scratch_shapes=[
            pltpu.VMEM((N_DEV, 2, m), jnp.float32),
            pltpu.SemaphoreType.DMA((N_DEV - 1,)),
            pltpu.SemaphoreType.DMA((N_DEV - 1,)),
        ],
        compiler_params=pltpu.CompilerParams(collective_id=0),
    )(x, gamma.reshape(1, n_per), beta.reshape(1, n_per))
